# baseline (speedup 1.0000x reference)
import jax
import jax.numpy as jnp
from jax.experimental import pallas as pl

_B = 16384
_K = 64
_BLK = 16384


def _rowdot_body(gu_ref, gi_ref, xui_ref, guo_ref, gio_ref):
    u = gu_ref[...]
    v = gi_ref[...]
    xui_ref[...] = jnp.sum(u * v, axis=0)
    guo_ref[...] = u
    gio_ref[...] = v


def kernel(gu, gi):
    gut = gu.T
    git = gi.T
    xui, guo_t, gio_t = pl.pallas_call(
        _rowdot_body,
        grid=(_B // _BLK,),
        in_specs=[
            pl.BlockSpec((_K, _BLK), lambda i: (0, i)),
            pl.BlockSpec((_K, _BLK), lambda i: (0, i)),
        ],
        out_specs=[
            pl.BlockSpec((_BLK,), lambda i: (i,)),
            pl.BlockSpec((_K, _BLK), lambda i: (0, i)),
            pl.BlockSpec((_K, _BLK), lambda i: (0, i)),
        ],
        out_shape=[
            jax.ShapeDtypeStruct((_B,), jnp.float32),
            jax.ShapeDtypeStruct((_K, _B), jnp.float32),
            jax.ShapeDtypeStruct((_K, _B), jnp.float32),
        ],
    )(gut, git)
    return (xui, guo_t.T, gio_t.T)
